# baseline (speedup 1.0000x reference)
import jax
import jax.numpy as jnp
from jax import lax
from jax.experimental import pallas as pl
from jax.experimental.pallas import tpu as pltpu
from jax.experimental.pallas import tpu_sc as plsc

_N_ORB = 19
_MAX_E = 15
_D = 128
_N_ELEM = 96
_N_ATOMS = 10000

_NC = 2
_NS = 16
_NW = _NC * _NS

_C = 16
_BPW = 312
_NCH = 20
_EPW = _N_ELEM // _NS
_CI_PAD = 24


def _sc_body(z_hbm, ci_hbm, tabs_hbm, out_hbm, comb_hbm,
             idx_v, rows_a, zb0, rb0, zb1, rb1, sem_a, sem0, sem1):
    c = lax.axis_index("c")
    s = lax.axis_index("s")
    wid = s * _NC + c

    for j in range(_EPW):
        e = s * _EPW + j
        pltpu.sync_copy(ci_hbm.at[e], idx_v)
        pltpu.async_copy(tabs_hbm.at[idx_v], rows_a, sem_a).wait()
        pltpu.sync_copy(rows_a.at[pl.ds(0, _N_ORB)], comb_hbm.at[e])
    plsc.subcore_barrier()

    base = wid * _BPW
    wend = base + _BPW + jnp.where(wid == _NW - 1, _N_ATOMS - _NW * _BPW, 0)
    slots = ((zb0, rb0, sem0), (zb1, rb1, sem1))

    def cb(k):
        return jnp.minimum(base + k * _C, wend - _C)

    def start(k):
        zb, rb, sem = slots[k % 2]
        pltpu.sync_copy(z_hbm.at[pl.ds(cb(k), _C)], zb)
        pltpu.async_copy(comb_hbm.at[zb], rb, sem)

    def finish(k):
        zb, rb, sem = slots[k % 2]
        pltpu.make_async_copy(comb_hbm.at[zb], rb, sem).wait()
        pltpu.sync_copy(rb, out_hbm.at[pl.ds(cb(k), _C)])

    start(0)
    for k in range(_NCH):
        if k + 1 < _NCH:
            start(k + 1)
        else:
            @pl.when(wid == _NW - 1)
            def _():
                start(_NCH)
        finish(k)

    @pl.when(wid == _NW - 1)
    def _():
        finish(_NCH)


def kernel(z, elec_table, tables):
    z = z.astype(jnp.int32)
    tabs = tables.at[:, 0, :].set(0.0).reshape(_N_ORB * _MAX_E, _D)
    ci = elec_table.astype(jnp.int32) + (jnp.arange(_N_ORB, dtype=jnp.int32) * _MAX_E)[None, :]
    ci = jnp.pad(ci, ((0, 0), (0, _CI_PAD - _N_ORB)))

    mesh = plsc.VectorSubcoreMesh(core_axis_name="c", subcore_axis_name="s")
    out, _ = pl.kernel(
        _sc_body,
        out_type=[
            jax.ShapeDtypeStruct((_N_ATOMS, _N_ORB, _D), jnp.float32),
            jax.ShapeDtypeStruct((_N_ELEM, _N_ORB, _D), jnp.float32),
        ],
        mesh=mesh,
        scratch_types=[
            pltpu.VMEM((_CI_PAD,), jnp.int32),
            pltpu.VMEM((_CI_PAD, _D), jnp.float32),
            pltpu.VMEM((_C,), jnp.int32),
            pltpu.VMEM((_C, _N_ORB, _D), jnp.float32),
            pltpu.VMEM((_C,), jnp.int32),
            pltpu.VMEM((_C, _N_ORB, _D), jnp.float32),
            pltpu.SemaphoreType.DMA,
            pltpu.SemaphoreType.DMA,
            pltpu.SemaphoreType.DMA,
        ],
    )(z, ci, tabs)
    return out
